# baseline (speedup 1.0000x reference)
import functools

import jax
import jax.numpy as jnp
from jax import lax
from jax.experimental import pallas as pl
from jax.experimental.pallas import tpu as pltpu
from jax.experimental.pallas import tpu_sc as plsc

N = 4096
PRE = 1536
POST = 768
HW = 4096
SIGMA = 0.5
THRESH = 0.5
NC = 2
NS = 16
NW = NC * NS
PRE_W = PRE // NW
POST_W = POST // NW
ROWS_CHUNK = 24

RB = 512


def _ranks_body(s_col_ref, s_row_ref, o_ref):
    r = pl.program_id(0)
    s_col = s_col_ref[:]
    s_row = s_row_ref[:]
    gt = s_row > s_col
    jj = lax.broadcasted_iota(jnp.int32, (RB, N), 1)
    ii = lax.broadcasted_iota(jnp.int32, (RB, N), 0) + r * RB
    eq_lt = (s_row == s_col) & (jj < ii)
    o_ref[:] = jnp.sum((gt | eq_lt).astype(jnp.int32), axis=1, keepdims=True)


def _ranks_tc(scores):
    out = pl.pallas_call(
        _ranks_body,
        grid=(N // RB,),
        in_specs=[
            pl.BlockSpec((RB, 1), lambda r: (r, 0)),
            pl.BlockSpec((1, N), lambda r: (0, 0)),
        ],
        out_specs=pl.BlockSpec((RB, 1), lambda r: (r, 0)),
        out_shape=jax.ShapeDtypeStruct((N, 1), jnp.int32),
    )(scores.reshape(N, 1), scores.reshape(1, N))
    return out.reshape(N)


def _gather_sc(ranks, scores, labels, factors, msum, bm2):
    mesh = plsc.VectorSubcoreMesh(core_axis_name="c", subcore_axis_name="s")

    @functools.partial(
        pl.kernel,
        mesh=mesh,
        out_type=(
            jax.ShapeDtypeStruct((PRE,), jnp.int32),
            jax.ShapeDtypeStruct((PRE,), jnp.float32),
            jax.ShapeDtypeStruct((PRE,), jnp.int32),
            jax.ShapeDtypeStruct((PRE,), jnp.float32),
            jax.ShapeDtypeStruct((PRE,), jnp.float32),
            jax.ShapeDtypeStruct((PRE, HW), jnp.float32),
        ),
        scratch_types=(
            pltpu.VMEM((N,), jnp.int32),
            pltpu.VMEM((N,), jnp.float32),
            pltpu.VMEM((N,), jnp.int32),
            pltpu.VMEM((N,), jnp.float32),
            pltpu.VMEM((N,), jnp.float32),
            pltpu.VMEM((PRE_W,), jnp.int32),
            pltpu.VMEM((PRE_W,), jnp.float32),
            pltpu.VMEM((PRE_W,), jnp.int32),
            pltpu.VMEM((PRE_W,), jnp.float32),
            pltpu.VMEM((PRE_W,), jnp.float32),
            pltpu.VMEM((ROWS_CHUNK, HW), jnp.float32),
            pltpu.SemaphoreType.DMA,
        ),
        compiler_params=pltpu.CompilerParams(needs_layout_passes=False),
    )
    def k(ranks_hbm, s_hbm, l_hbm, f_hbm, m_hbm, bm_hbm,
          idx_hbm, sc_hbm, lab_hbm, cf_hbm, ms_hbm, bmsel_hbm,
          ranks_v, tab_s, tab_l, tab_f, tab_m,
          idx48, sc48, lab48, cf48, ms48, rows_v, sem):
        wid = lax.axis_index("s") * NC + lax.axis_index("c")
        base = wid * PRE_W
        pltpu.sync_copy(ranks_hbm, ranks_v)
        pltpu.sync_copy(s_hbm, tab_s)
        pltpu.sync_copy(l_hbm, tab_l)
        pltpu.sync_copy(f_hbm, tab_f)
        pltpu.sync_copy(m_hbm, tab_m)
        for k16 in range(PRE_W // 16):
            idx48[pl.ds(k16 * 16, 16)] = jnp.zeros((16,), jnp.int32)

        @pl.loop(0, N // 16, unroll=8)
        def _(t):
            r = ranks_v[pl.ds(t * 16, 16)]
            m = (r >= base) & (r < base + PRE_W)
            lp = jnp.clip(r - base, 0, PRE_W - 1)
            vals = lax.iota(jnp.int32, 16) + t * 16
            plsc.store_scatter(idx48, [lp], vals, mask=m)

        for k16 in range(PRE_W // 16):
            sl = pl.ds(k16 * 16, 16)
            iv = idx48[sl]
            sc48[sl] = plsc.load_gather(tab_s, [iv])
            lab48[sl] = plsc.load_gather(tab_l, [iv])
            cf48[sl] = plsc.load_gather(tab_f, [iv])
            ms48[sl] = plsc.load_gather(tab_m, [iv])
        pltpu.sync_copy(idx48, idx_hbm.at[pl.ds(base, PRE_W)])
        pltpu.sync_copy(sc48, sc_hbm.at[pl.ds(base, PRE_W)])
        pltpu.sync_copy(lab48, lab_hbm.at[pl.ds(base, PRE_W)])
        pltpu.sync_copy(cf48, cf_hbm.at[pl.ds(base, PRE_W)])
        pltpu.sync_copy(ms48, ms_hbm.at[pl.ds(base, PRE_W)])
        for c in range(PRE_W // ROWS_CHUNK):
            pltpu.async_copy(
                bm_hbm.at[idx48.at[pl.ds(c * ROWS_CHUNK, ROWS_CHUNK)]],
                rows_v, sem).wait()
            pltpu.sync_copy(
                rows_v, bmsel_hbm.at[pl.ds(base + c * ROWS_CHUNK, ROWS_CHUNK)])

    return k(ranks, scores, labels, factors, msum, bm2)


def _nms_body(bm_ref, sc_ref, labr_ref, labc_ref, msr_ref, msc_ref, pos_ref):
    bm16 = bm_ref[:].astype(jnp.bfloat16)
    inter = lax.dot_general(bm16, bm16, (((1,), (1,)), ((), ())),
                            preferred_element_type=jnp.float32)
    ii = lax.broadcasted_iota(jnp.int32, (PRE, PRE), 0)
    jj = lax.broadcasted_iota(jnp.int32, (PRE, PRE), 1)
    match = labc_ref[:] == labr_ref[:]
    msr = msr_ref[:]
    msc = msc_ref[:]
    d = jnp.where(match & (ii < jj), inter / (msr + msr - inter + 1e-6), 0.0)
    g = jnp.where(match & (jj < ii), inter / (msc + msc - inter + 1e-6), 0.0)
    comp = jnp.max(g, axis=1, keepdims=True)
    mx = jnp.max(d * d - comp * comp, axis=0, keepdims=True)
    decayed = sc_ref[:] * jnp.exp(mx * (-1.0 / SIGMA))
    keep = decayed >= THRESH
    kf = keep.astype(jnp.bfloat16)
    tri = (ii <= jj).astype(jnp.bfloat16)
    cum = lax.dot_general(kf, tri, (((1,), (0,)), ((), ())),
                          preferred_element_type=jnp.float32)
    cumi = cum.astype(jnp.int32)
    nk = lax.slice(cumi, (0, PRE - 1), (1, PRE))
    ji = lax.broadcasted_iota(jnp.int32, (1, PRE), 1)
    pos_ref[:] = jnp.where(keep, cumi - 1, nk + ji - cumi)


def _nms_tc(bmsel, sc, lab, ms):
    pos = pl.pallas_call(
        _nms_body,
        out_shape=jax.ShapeDtypeStruct((1, PRE), jnp.int32),
    )(bmsel, sc.reshape(1, PRE), lab.reshape(1, PRE), lab.reshape(PRE, 1),
      ms.reshape(1, PRE), ms.reshape(PRE, 1))
    return pos.reshape(PRE)


def _final_sc(pos, sc, lab, cf, idx):
    mesh = plsc.VectorSubcoreMesh(core_axis_name="c", subcore_axis_name="s")

    @functools.partial(
        pl.kernel,
        mesh=mesh,
        out_type=(
            jax.ShapeDtypeStruct((POST,), jnp.int32),
            jax.ShapeDtypeStruct((POST,), jnp.float32),
            jax.ShapeDtypeStruct((POST,), jnp.int32),
            jax.ShapeDtypeStruct((POST,), jnp.float32),
        ),
        scratch_types=(
            pltpu.VMEM((PRE,), jnp.int32),
            pltpu.VMEM((PRE,), jnp.float32),
            pltpu.VMEM((PRE,), jnp.int32),
            pltpu.VMEM((PRE,), jnp.float32),
            pltpu.VMEM((PRE,), jnp.int32),
            pltpu.VMEM((32,), jnp.int32),
            pltpu.VMEM((32,), jnp.float32),
            pltpu.VMEM((32,), jnp.int32),
            pltpu.VMEM((32,), jnp.float32),
        ),
        compiler_params=pltpu.CompilerParams(needs_layout_passes=False),
    )
    def k(pos_hbm, sc_hbm, lab_hbm, cf_hbm, idx_hbm,
          fidx_hbm, osc_hbm, olab_hbm, ocf_hbm,
          pos_v, sc_v, lab_v, cf_v, idx_v,
          fidx, osc, olab, ocf):
        wid = lax.axis_index("s") * NC + lax.axis_index("c")
        base = wid * POST_W
        pltpu.sync_copy(pos_hbm, pos_v)
        pltpu.sync_copy(sc_hbm, sc_v)
        pltpu.sync_copy(lab_hbm, lab_v)
        pltpu.sync_copy(cf_hbm, cf_v)
        pltpu.sync_copy(idx_hbm, idx_v)
        for k16 in range(2):
            sl = pl.ds(k16 * 16, 16)
            fidx[sl] = jnp.zeros((16,), jnp.int32)
            osc[sl] = jnp.zeros((16,), jnp.float32)
            olab[sl] = jnp.zeros((16,), jnp.int32)
            ocf[sl] = jnp.zeros((16,), jnp.float32)

        @pl.loop(0, PRE // 16, unroll=8)
        def _(t):
            sl = pl.ds(t * 16, 16)
            p = pos_v[sl]
            m = (p >= base) & (p < base + POST_W)
            lp = jnp.clip(p - base, 0, POST_W - 1)
            plsc.store_scatter(fidx, [lp], idx_v[sl], mask=m)
            plsc.store_scatter(osc, [lp], sc_v[sl], mask=m)
            plsc.store_scatter(olab, [lp], lab_v[sl], mask=m)
            plsc.store_scatter(ocf, [lp], cf_v[sl], mask=m)

        pltpu.sync_copy(fidx.at[pl.ds(0, POST_W)], fidx_hbm.at[pl.ds(base, POST_W)])
        pltpu.sync_copy(osc.at[pl.ds(0, POST_W)], osc_hbm.at[pl.ds(base, POST_W)])
        pltpu.sync_copy(olab.at[pl.ds(0, POST_W)], olab_hbm.at[pl.ds(base, POST_W)])
        pltpu.sync_copy(ocf.at[pl.ds(0, POST_W)], ocf_hbm.at[pl.ds(base, POST_W)])

    return k(pos, sc, lab, cf, idx)


SEG_K = 8


def _seg_body(idx_ref, *refs):
    del idx_ref
    ins, out_ref = refs[:-1], refs[-1]
    for k in range(SEG_K):
        out_ref[k] = ins[k][0]


def _seg_gather_tc(fidx, seg3):
    grid_spec = pltpu.PrefetchScalarGridSpec(
        num_scalar_prefetch=1,
        grid=(POST // SEG_K,),
        in_specs=[
            pl.BlockSpec((1, 64, 64),
                         (lambda i, idx_ref, k=k: (idx_ref[i * SEG_K + k], 0, 0)))
            for k in range(SEG_K)
        ],
        out_specs=pl.BlockSpec((SEG_K, 64, 64), lambda i, idx_ref: (i, 0, 0)),
    )
    return pl.pallas_call(
        _seg_body,
        grid_spec=grid_spec,
        out_shape=jax.ShapeDtypeStruct((POST, 64, 64), jnp.float32),
    )(fidx, *([seg3] * SEG_K))


def kernel(cls_labels, scores, cls_factors, seg_preds, binary_masks, mask_sum):
    bm2 = binary_masks.reshape(N, HW)
    ranks = _ranks_tc(scores)
    idx, sc, lab, cf, ms, bmsel = _gather_sc(
        ranks, scores, cls_labels, cls_factors, mask_sum, bm2)
    pos = _nms_tc(bmsel, sc, lab, ms)
    fidx, osc, olab, ocf = _final_sc(pos, sc, lab, cf, idx)
    oseg = _seg_gather_tc(fidx, seg_preds)
    return oseg, osc, olab, ocf


# speedup vs baseline: 1.6600x; 1.6600x over previous
import functools

import jax
import jax.numpy as jnp
from jax import lax
from jax.experimental import pallas as pl
from jax.experimental.pallas import tpu as pltpu
from jax.experimental.pallas import tpu_sc as plsc

N = 4096
PRE = 1536
POST = 768
HW = 4096
SIGMA = 0.5
THRESH = 0.5
NC = 2
NS = 16
NW = NC * NS
PRE_W = PRE // NW
POST_W = POST // NW
ROWS_CHUNK = 24

RB = 512


def _ranks_body(s_col_ref, s_row_ref, o_ref):
    r = pl.program_id(0)
    s_col = s_col_ref[:]
    s_row = s_row_ref[:]
    gt = s_row > s_col
    jj = lax.broadcasted_iota(jnp.int32, (RB, N), 1)
    ii = lax.broadcasted_iota(jnp.int32, (RB, N), 0) + r * RB
    eq_lt = (s_row == s_col) & (jj < ii)
    o_ref[:] = jnp.sum((gt | eq_lt).astype(jnp.int32), axis=1, keepdims=True)


def _ranks_tc(scores):
    out = pl.pallas_call(
        _ranks_body,
        grid=(N // RB,),
        in_specs=[
            pl.BlockSpec((RB, 1), lambda r: (r, 0)),
            pl.BlockSpec((1, N), lambda r: (0, 0)),
        ],
        out_specs=pl.BlockSpec((RB, 1), lambda r: (r, 0)),
        out_shape=jax.ShapeDtypeStruct((N, 1), jnp.int32),
    )(scores.reshape(N, 1), scores.reshape(1, N))
    return out.reshape(N)


def _gather_sc(ranks, scores, labels, factors, msum):
    mesh = plsc.VectorSubcoreMesh(core_axis_name="c", subcore_axis_name="s")

    @functools.partial(
        pl.kernel,
        mesh=mesh,
        out_type=(
            jax.ShapeDtypeStruct((PRE,), jnp.int32),
            jax.ShapeDtypeStruct((PRE,), jnp.float32),
            jax.ShapeDtypeStruct((PRE,), jnp.int32),
            jax.ShapeDtypeStruct((PRE,), jnp.float32),
            jax.ShapeDtypeStruct((PRE,), jnp.float32),
        ),
        scratch_types=(
            pltpu.VMEM((N,), jnp.int32),
            pltpu.VMEM((N,), jnp.float32),
            pltpu.VMEM((N,), jnp.int32),
            pltpu.VMEM((N,), jnp.float32),
            pltpu.VMEM((N,), jnp.float32),
            pltpu.VMEM((PRE_W,), jnp.int32),
            pltpu.VMEM((PRE_W,), jnp.float32),
            pltpu.VMEM((PRE_W,), jnp.int32),
            pltpu.VMEM((PRE_W,), jnp.float32),
            pltpu.VMEM((PRE_W,), jnp.float32),
        ),
        compiler_params=pltpu.CompilerParams(needs_layout_passes=False),
    )
    def k(ranks_hbm, s_hbm, l_hbm, f_hbm, m_hbm,
          idx_hbm, sc_hbm, lab_hbm, cf_hbm, ms_hbm,
          ranks_v, tab_s, tab_l, tab_f, tab_m,
          idx48, sc48, lab48, cf48, ms48):
        wid = lax.axis_index("s") * NC + lax.axis_index("c")
        base = wid * PRE_W
        pltpu.sync_copy(ranks_hbm, ranks_v)
        pltpu.sync_copy(s_hbm, tab_s)
        pltpu.sync_copy(l_hbm, tab_l)
        pltpu.sync_copy(f_hbm, tab_f)
        pltpu.sync_copy(m_hbm, tab_m)
        for k16 in range(PRE_W // 16):
            idx48[pl.ds(k16 * 16, 16)] = jnp.zeros((16,), jnp.int32)

        @pl.loop(0, N // 16, unroll=8)
        def _(t):
            r = ranks_v[pl.ds(t * 16, 16)]
            m = (r >= base) & (r < base + PRE_W)
            lp = jnp.clip(r - base, 0, PRE_W - 1)
            vals = lax.iota(jnp.int32, 16) + t * 16
            plsc.store_scatter(idx48, [lp], vals, mask=m)

        for k16 in range(PRE_W // 16):
            sl = pl.ds(k16 * 16, 16)
            iv = idx48[sl]
            sc48[sl] = plsc.load_gather(tab_s, [iv])
            lab48[sl] = plsc.load_gather(tab_l, [iv])
            cf48[sl] = plsc.load_gather(tab_f, [iv])
            ms48[sl] = plsc.load_gather(tab_m, [iv])
        pltpu.sync_copy(idx48, idx_hbm.at[pl.ds(base, PRE_W)])
        pltpu.sync_copy(sc48, sc_hbm.at[pl.ds(base, PRE_W)])
        pltpu.sync_copy(lab48, lab_hbm.at[pl.ds(base, PRE_W)])
        pltpu.sync_copy(cf48, cf_hbm.at[pl.ds(base, PRE_W)])
        pltpu.sync_copy(ms48, ms_hbm.at[pl.ds(base, PRE_W)])

    return k(ranks, scores, labels, factors, msum)


NBLK = 8
PB = HW // NBLK


def _nms_body(bmT_ref, idx_ref, sc_ref, labr_ref, labc_ref, msr_ref, msc_ref,
              pos_ref, acc_ref, h_ref):
    i = pl.program_id(0)

    @pl.when(i == 0)
    def _():
        kk = lax.broadcasted_iota(jnp.int32, (N, PRE), 0)
        h_ref[:] = (kk == idx_ref[:]).astype(jnp.bfloat16)

    blk16 = bmT_ref[:].astype(jnp.bfloat16)
    a = lax.dot_general(blk16, h_ref[:], (((1,), (0,)), ((), ())),
                        preferred_element_type=jnp.float32)
    a16 = a.astype(jnp.bfloat16)
    contrib = lax.dot_general(a16, a16, (((0,), (0,)), ((), ())),
                              preferred_element_type=jnp.float32)

    @pl.when(i == 0)
    def _():
        acc_ref[:] = contrib

    @pl.when(i > 0)
    def _():
        acc_ref[:] += contrib

    @pl.when(i == NBLK - 1)
    def _():
        inter = acc_ref[:]
        ii = lax.broadcasted_iota(jnp.int32, (PRE, PRE), 0)
        jj = lax.broadcasted_iota(jnp.int32, (PRE, PRE), 1)
        match = labc_ref[:] == labr_ref[:]
        msr = msr_ref[:]
        msc = msc_ref[:]
        d = jnp.where(match & (ii < jj), inter / (msr + msr - inter + 1e-6), 0.0)
        g = jnp.where(match & (jj < ii), inter / (msc + msc - inter + 1e-6), 0.0)
        comp = jnp.max(g, axis=1, keepdims=True)
        mx = jnp.max(d * d - comp * comp, axis=0, keepdims=True)
        decayed = sc_ref[:] * jnp.exp(mx * (-1.0 / SIGMA))
        keep = decayed >= THRESH
        kf = keep.astype(jnp.bfloat16)
        tri = (ii <= jj).astype(jnp.bfloat16)
        cum = lax.dot_general(kf, tri, (((1,), (0,)), ((), ())),
                              preferred_element_type=jnp.float32)
        cumi = cum.astype(jnp.int32)
        nk = lax.slice(cumi, (0, PRE - 1), (1, PRE))
        ji = lax.broadcasted_iota(jnp.int32, (1, PRE), 1)
        pos_ref[:] = jnp.where(keep, cumi - 1, nk + ji - cumi)


def _nms_tc(bmT, idx, sc, lab, ms):
    pos = pl.pallas_call(
        _nms_body,
        grid=(NBLK,),
        in_specs=[
            pl.BlockSpec((PB, N), lambda i: (i, 0)),
            pl.BlockSpec((1, PRE), lambda i: (0, 0)),
            pl.BlockSpec((1, PRE), lambda i: (0, 0)),
            pl.BlockSpec((1, PRE), lambda i: (0, 0)),
            pl.BlockSpec((PRE, 1), lambda i: (0, 0)),
            pl.BlockSpec((1, PRE), lambda i: (0, 0)),
            pl.BlockSpec((PRE, 1), lambda i: (0, 0)),
        ],
        out_specs=pl.BlockSpec((1, PRE), lambda i: (0, 0)),
        out_shape=jax.ShapeDtypeStruct((1, PRE), jnp.int32),
        scratch_shapes=[pltpu.VMEM((PRE, PRE), jnp.float32),
                        pltpu.VMEM((N, PRE), jnp.bfloat16)],
    )(bmT, idx.reshape(1, PRE), sc.reshape(1, PRE), lab.reshape(1, PRE),
      lab.reshape(PRE, 1), ms.reshape(1, PRE), ms.reshape(PRE, 1))
    return pos.reshape(PRE)


def _final_sc(pos, sc, lab, cf, idx):
    mesh = plsc.VectorSubcoreMesh(core_axis_name="c", subcore_axis_name="s")

    @functools.partial(
        pl.kernel,
        mesh=mesh,
        out_type=(
            jax.ShapeDtypeStruct((POST,), jnp.int32),
            jax.ShapeDtypeStruct((POST,), jnp.float32),
            jax.ShapeDtypeStruct((POST,), jnp.int32),
            jax.ShapeDtypeStruct((POST,), jnp.float32),
        ),
        scratch_types=(
            pltpu.VMEM((PRE,), jnp.int32),
            pltpu.VMEM((PRE,), jnp.float32),
            pltpu.VMEM((PRE,), jnp.int32),
            pltpu.VMEM((PRE,), jnp.float32),
            pltpu.VMEM((PRE,), jnp.int32),
            pltpu.VMEM((32,), jnp.int32),
            pltpu.VMEM((32,), jnp.float32),
            pltpu.VMEM((32,), jnp.int32),
            pltpu.VMEM((32,), jnp.float32),
        ),
        compiler_params=pltpu.CompilerParams(needs_layout_passes=False),
    )
    def k(pos_hbm, sc_hbm, lab_hbm, cf_hbm, idx_hbm,
          fidx_hbm, osc_hbm, olab_hbm, ocf_hbm,
          pos_v, sc_v, lab_v, cf_v, idx_v,
          fidx, osc, olab, ocf):
        wid = lax.axis_index("s") * NC + lax.axis_index("c")
        base = wid * POST_W
        pltpu.sync_copy(pos_hbm, pos_v)
        pltpu.sync_copy(sc_hbm, sc_v)
        pltpu.sync_copy(lab_hbm, lab_v)
        pltpu.sync_copy(cf_hbm, cf_v)
        pltpu.sync_copy(idx_hbm, idx_v)
        for k16 in range(2):
            sl = pl.ds(k16 * 16, 16)
            fidx[sl] = jnp.zeros((16,), jnp.int32)
            osc[sl] = jnp.zeros((16,), jnp.float32)
            olab[sl] = jnp.zeros((16,), jnp.int32)
            ocf[sl] = jnp.zeros((16,), jnp.float32)

        @pl.loop(0, PRE // 16, unroll=8)
        def _(t):
            sl = pl.ds(t * 16, 16)
            p = pos_v[sl]
            m = (p >= base) & (p < base + POST_W)
            lp = jnp.clip(p - base, 0, POST_W - 1)
            plsc.store_scatter(fidx, [lp], idx_v[sl], mask=m)
            plsc.store_scatter(osc, [lp], sc_v[sl], mask=m)
            plsc.store_scatter(olab, [lp], lab_v[sl], mask=m)
            plsc.store_scatter(ocf, [lp], cf_v[sl], mask=m)

        pltpu.sync_copy(fidx.at[pl.ds(0, POST_W)], fidx_hbm.at[pl.ds(base, POST_W)])
        pltpu.sync_copy(osc.at[pl.ds(0, POST_W)], osc_hbm.at[pl.ds(base, POST_W)])
        pltpu.sync_copy(olab.at[pl.ds(0, POST_W)], olab_hbm.at[pl.ds(base, POST_W)])
        pltpu.sync_copy(ocf.at[pl.ds(0, POST_W)], ocf_hbm.at[pl.ds(base, POST_W)])

    return k(pos, sc, lab, cf, idx)


def _seg_body(segT_ref, fidx_ref, out_ref, h_ref):
    i = pl.program_id(0)

    @pl.when(i == 0)
    def _():
        kk = lax.broadcasted_iota(jnp.int32, (N, POST), 0)
        h_ref[:] = (kk == fidx_ref[:]).astype(jnp.bfloat16)

    blk = segT_ref[:]
    hi = blk.astype(jnp.bfloat16)
    lo = (blk - hi.astype(jnp.float32)).astype(jnp.bfloat16)
    dims = (((1,), (0,)), ((), ()))
    out_ref[:] = (
        lax.dot_general(hi, h_ref[:], dims, preferred_element_type=jnp.float32)
        + lax.dot_general(lo, h_ref[:], dims, preferred_element_type=jnp.float32))


def _seg_gather_tc(segT, fidx):
    return pl.pallas_call(
        _seg_body,
        grid=(NBLK,),
        in_specs=[
            pl.BlockSpec((PB, N), lambda i: (i, 0)),
            pl.BlockSpec((1, POST), lambda i: (0, 0)),
        ],
        out_specs=pl.BlockSpec((PB, POST), lambda i: (i, 0)),
        out_shape=jax.ShapeDtypeStruct((HW, POST), jnp.float32),
        scratch_shapes=[pltpu.VMEM((N, POST), jnp.bfloat16)],
    )(segT, fidx.reshape(1, POST))


def kernel(cls_labels, scores, cls_factors, seg_preds, binary_masks, mask_sum):
    bmT = binary_masks.transpose(1, 2, 0).reshape(HW, N)
    segT = seg_preds.transpose(1, 2, 0).reshape(HW, N)
    ranks = _ranks_tc(scores)
    idx, sc, lab, cf, ms = _gather_sc(
        ranks, scores, cls_labels, cls_factors, mask_sum)
    pos = _nms_tc(bmT, idx, sc, lab, ms)
    fidx, osc, olab, ocf = _final_sc(pos, sc, lab, cf, idx)
    osegT = _seg_gather_tc(segT, fidx)
    oseg = osegT.reshape(64, 64, POST).transpose(2, 0, 1)
    return oseg, osc, olab, ocf
